# Initial kernel scaffold; baseline (speedup 1.0000x reference)
import jax
import jax.numpy as jnp
from jax.experimental import pallas as pl


def kernel(player_loc, player_conf, player_loc_t, player_conf_t):
    raise NotImplementedError("write your pallas kernel here")


# speedup vs baseline: 9.4321x; 9.4321x over previous
import functools

import jax
import jax.numpy as jnp
from jax import lax
from jax.experimental import pallas as pl
from jax.experimental.pallas import tpu as pltpu

_B = 128
_P = 20000
_RATIO = 3
_S = _P // 32


def _body(labels_ref, conf_ref, loc_ref, loct_ref,
          o_numpos, o_cepos, o_topsum, o_sl1, negloss_ref):
    labels = labels_ref[0]
    conf = conf_ref[0]
    loc = loc_ref[0]
    loct = loct_ref[0]

    posf = (labels > 0).astype(jnp.float32)

    lane64 = lax.broadcasted_iota(jnp.int32, (64, 32), 0)
    col32 = lax.broadcasted_iota(jnp.int32, (64, 32), 1)
    s0 = (lane64 == 2 * col32).astype(jnp.float32)
    s1 = (lane64 == 2 * col32 + 1).astype(jnp.float32)
    c0 = jnp.dot(conf, s0, preferred_element_type=jnp.float32)
    c1 = jnp.dot(conf, s1, preferred_element_type=jnp.float32)

    d = c1 - c0
    l1p = jnp.log1p(jnp.exp(-jnp.abs(d)))
    loss_neg = jnp.maximum(d, 0.0) + l1p
    ce_pos = jnp.maximum(-d, 0.0) + l1p

    num_pos_f = jnp.sum(posf)
    sum_ce_pos = jnp.sum(posf * ce_pos)
    negf = 1.0 - posf
    sum_neg_loss = jnp.sum(negf * loss_neg)

    neg_vals = jnp.where(posf > 0.0, -1.0, loss_neg)
    negloss_ref[...] = neg_vals

    dd = loc - loct
    ad = jnp.abs(dd)
    sl1 = jnp.where(ad < 1.0, 0.5 * dd * dd, ad - 0.5)
    lane128 = lax.broadcasted_iota(jnp.int32, (32, 128), 1)
    row32 = lax.broadcasted_iota(jnp.int32, (32, 128), 0)
    rexp = (lane128 // 4 == row32).astype(jnp.float32)
    mask4 = jnp.dot(posf, rexp, preferred_element_type=jnp.float32)
    sum_sl1 = jnp.sum(mask4 * sl1)

    num_pos_i = num_pos_f.astype(jnp.int32)
    k = _RATIO * jnp.maximum(num_pos_i, 1)
    neg_count = _P - num_pos_i

    o_numpos[...] = jnp.broadcast_to(num_pos_f, (1, 1, 128))
    o_cepos[...] = jnp.broadcast_to(sum_ce_pos, (1, 1, 128))
    o_sl1[...] = jnp.broadcast_to(sum_sl1, (1, 1, 128))
    o_topsum[...] = jnp.broadcast_to(sum_neg_loss, (1, 1, 128))

    @pl.when(k < neg_count)
    def _select():
        kf = k.astype(jnp.float32)
        vals = negloss_ref[...]

        def step(_, carry):
            lo, hi = carry
            mid = lo + (hi - lo) // 2
            thr = lax.bitcast_convert_type(mid, jnp.float32)
            cnt = jnp.sum((vals >= thr).astype(jnp.float32))
            take = cnt >= kf
            return (jnp.where(take, mid, lo), jnp.where(take, hi, mid))

        lo0 = jnp.int32(0)
        hi0 = jnp.int32(0x7F800000)
        lo, _ = lax.fori_loop(0, 31, step, (lo0, hi0))
        t = lax.bitcast_convert_type(lo, jnp.float32)
        gt = vals > t
        cnt_gt = jnp.sum(gt.astype(jnp.float32))
        sum_gt = jnp.sum(jnp.where(gt, vals, 0.0))
        top = sum_gt + (kf - cnt_gt) * t
        o_topsum[...] = jnp.broadcast_to(top, (1, 1, 128))


def kernel(player_loc, player_conf, player_loc_t, player_conf_t):
    labels_r = player_conf_t.reshape(_B, _S, 32)
    conf_r = player_conf.reshape(_B, _S, 64)
    loc_r = player_loc.reshape(_B, _S, 128)
    loct_r = player_loc_t.reshape(_B, _S, 128)

    out_sds = jax.ShapeDtypeStruct((_B, 1, 128), jnp.float32)
    o_np, o_ce, o_top, o_sl1 = pl.pallas_call(
        _body,
        grid=(_B,),
        in_specs=[
            pl.BlockSpec((1, _S, 32), lambda r: (r, 0, 0)),
            pl.BlockSpec((1, _S, 64), lambda r: (r, 0, 0)),
            pl.BlockSpec((1, _S, 128), lambda r: (r, 0, 0)),
            pl.BlockSpec((1, _S, 128), lambda r: (r, 0, 0)),
        ],
        out_specs=[
            pl.BlockSpec((1, 1, 128), lambda r: (r, 0, 0)),
            pl.BlockSpec((1, 1, 128), lambda r: (r, 0, 0)),
            pl.BlockSpec((1, 1, 128), lambda r: (r, 0, 0)),
            pl.BlockSpec((1, 1, 128), lambda r: (r, 0, 0)),
        ],
        out_shape=[out_sds, out_sds, out_sds, out_sds],
        scratch_shapes=[pltpu.VMEM((_S, 32), jnp.float32)],
    )(labels_r, conf_r, loc_r, loct_r)

    num_pos = o_np[:, 0, 0]
    num_pos_total = jnp.sum(jnp.maximum(num_pos, 1.0))
    loss_c = (jnp.sum(o_ce[:, 0, 0]) + jnp.sum(o_top[:, 0, 0])) / num_pos_total
    loss_l = jnp.sum(o_sl1[:, 0, 0]) / num_pos_total
    return (loss_l, loss_c)
